# baseline (speedup 1.0000x reference)
import functools

import jax
import jax.numpy as jnp
from jax import lax
from jax.experimental import pallas as pl
from jax.experimental.pallas import tpu as pltpu
from jax.experimental.pallas import tpu_sc as plsc

_EPS = 1e-5
_NC = 2
_NS = 16
_NW = _NC * _NS
_CH = 128
_GPB = 2
_BUF = _CH * _GPB


def _ln_body(t_ref, g_ref, b_ref, o_ref):
    t = t_ref[...]
    mean = jnp.mean(t, axis=1, keepdims=True)
    var = jnp.mean((t - mean) ** 2, axis=1, keepdims=True)
    o_ref[...] = (t - mean) * lax.rsqrt(var + _EPS) * g_ref[...] + b_ref[...]


def _make_sc_gather(n, d):
    rw = n // _NW
    steps = rw // _BUF
    mesh = plsc.VectorSubcoreMesh(core_axis_name="c", subcore_axis_name="s")

    @functools.partial(
        pl.kernel,
        out_type=jax.ShapeDtypeStruct((n, d), jnp.float32),
        mesh=mesh,
        scratch_types=[
            pltpu.VMEM((rw,), jnp.int32),
            pltpu.VMEM((_BUF, d), jnp.float32),
            pltpu.SemaphoreType.DMA,
        ],
    )
    def sc_gather(nt_hbm, idx_hbm, out_hbm, idx_v, rows_v, gsem):
        wid = lax.axis_index("s") * _NC + lax.axis_index("c")
        base = wid * rw
        pltpu.sync_copy(idx_hbm.at[pl.ds(base, rw)], idx_v)

        def body(step, carry):
            handles = []
            for j in range(_GPB):
                handles.append(pltpu.async_copy(
                    nt_hbm.at[idx_v.at[pl.ds(step * _BUF + j * _CH, _CH)]],
                    rows_v.at[pl.ds(j * _CH, _CH)],
                    gsem,
                ))
            for h in handles:
                h.wait()
            pltpu.sync_copy(rows_v, out_hbm.at[pl.ds(base + step * _BUF, _BUF)])
            return carry

        lax.fori_loop(0, steps, body, 0)

    return sc_gather


def kernel(grid, table, gamma, beta):
    batch, h, w = grid.shape
    v, d = table.shape
    n = batch * h * w

    tpad = jnp.zeros((16, d), jnp.float32).at[:v].set(table.astype(jnp.float32))
    nt = pl.pallas_call(
        _ln_body,
        out_shape=jax.ShapeDtypeStruct((16, d), jnp.float32),
    )(tpad, gamma.reshape(1, d), beta.reshape(1, d))

    idx = grid.reshape(n).astype(jnp.int32)
    out = _make_sc_gather(n, d)(nt, idx)
    return out.reshape(batch, h * w, d)
